# baseline (speedup 1.0000x reference)
import jax
import jax.numpy as jnp
from jax import lax
from jax.experimental import pallas as pl
from jax.experimental.pallas import tpu as pltpu
from jax.experimental.pallas import tpu_sc as plsc

N = 10000
E = 160000
D = 256
H = D // 2
EPS = 1e-5

NC = 2
NS = 16
NP = 10240
RPT = NP // NS
EPT = E // NS
K = 120
CHUNKS = EPT // K
TAIL = EPT - CHUNKS * K

DEG_E_SC = E // NC
DEG_E_T = DEG_E_SC // NS
DK = 128
DEG_CHUNKS = DEG_E_T // DK
DEG_TAIL = DEG_E_T - DEG_CHUNKS * DK

_mesh = plsc.VectorSubcoreMesh(core_axis_name="c", subcore_axis_name="s")
f32 = jnp.float32
i32 = jnp.int32


def _deg_body(dst_hbm, zeros_hbm, ones_hbm, degA_hbm, degB_hbm,
              acc, ones_v,
              didx0, didx1, didx2, didx3, didx4, didx5, didx6, didx7, tdidx,
              di0, di1, di2, di3, di4, di5, di6, di7,
              sc0, sc1, sc2, sc3):
    c = lax.axis_index("c")
    s = lax.axis_index("s")
    r0 = s * RPT
    base = c * DEG_E_SC + s * DEG_E_T
    didx = (didx0, didx1, didx2, didx3, didx4, didx5, didx6, didx7)
    di = (di0, di1, di2, di3, di4, di5, di6, di7)
    sc = (sc0, sc1, sc2, sc3)

    def issue_idx(g, q):
        off = pl.multiple_of(base + g * DK, 8)
        pltpu.async_copy(dst_hbm.at[pl.ds(off, DK)], didx[q], di[q])

    def wait_idx(q):
        pltpu.make_async_copy(dst_hbm.at[pl.ds(0, DK)], didx[q], di[q]).wait()

    def scatter(q8, q4):
        pltpu.async_copy(ones_v, acc.at[didx[q8]], sc[q4], add=True)

    def wait_scatter(q4):
        pltpu.make_async_copy(ones_v, acc.at[didx[0]], sc[q4]).wait()

    for q in range(4):
        issue_idx(q, q)
    pltpu.sync_copy(ones_hbm, ones_v)
    pltpu.sync_copy(zeros_hbm.at[pl.ds(r0, RPT)], acc.at[pl.ds(r0, RPT)])
    plsc.subcore_barrier()

    def octet(i, carry):
        for j in range(8):
            g = 8 * i + j
            q4 = j % 4

            @pl.when(g < DEG_CHUNKS)
            def _():
                wait_idx(j)

            @pl.when((g >= 4) & (g < DEG_CHUNKS))
            def _():
                wait_scatter(q4)

            @pl.when(g < DEG_CHUNKS)
            def _():
                scatter(j, q4)

            @pl.when(g + 4 < DEG_CHUNKS)
            def _():
                issue_idx(g + 4, (j + 4) % 8)

        return carry

    lax.fori_loop(0, (DEG_CHUNKS + 7) // 8, octet, 0)
    for q4 in range(4):
        wait_scatter(q4)
    toff = base + DEG_CHUNKS * DK
    pltpu.sync_copy(dst_hbm.at[pl.ds(toff, DEG_TAIL)], tdidx)
    pltpu.sync_copy(ones_v.at[pl.ds(0, DEG_TAIL)], acc.at[tdidx], add=True)
    plsc.subcore_barrier()

    @pl.when(c == 0)
    def _():
        pltpu.sync_copy(acc.at[pl.ds(r0, RPT)], degA_hbm.at[pl.ds(r0, RPT)])

    @pl.when(c == 1)
    def _():
        pltpu.sync_copy(acc.at[pl.ds(r0, RPT)], degB_hbm.at[pl.ds(r0, RPT)])


_deg_call = pl.kernel(
    _deg_body,
    out_type=[jax.ShapeDtypeStruct((NP, 128), f32)] * 2,
    mesh=_mesh,
    scratch_types=(
        [pltpu.VMEM_SHARED((NP, 128), f32)]
        + [pltpu.VMEM((DK, 128), f32)]
        + [pltpu.VMEM((DK,), i32)] * 8
        + [pltpu.VMEM((DEG_TAIL,), i32)]
        + [pltpu.SemaphoreType.DMA] * 12
    ),
)


def _agg_body(hA_hbm, hB_hbm, src_hbm, dst_hbm, outA_hbm, outB_hbm,
              acc, sidx0, sidx1, sidx2, didx0, didx1, didx2,
              rows0, rows1, rows2, tsidx, tdidx,
              si0, si1, si2, di0, di1, di2,
              g0, g1, g2, sc0, sc1, sc2):
    c = lax.axis_index("c")
    s = lax.axis_index("s")
    r0 = s * RPT
    base = s * EPT
    sidx = (sidx0, sidx1, sidx2)
    didx = (didx0, didx1, didx2)
    rows = (rows0, rows1, rows2)
    si = (si0, si1, si2)
    di = (di0, di1, di2)
    gs = (g0, g1, g2)
    sc = (sc0, sc1, sc2)

    def gather(q):
        @pl.when(c == 0)
        def _():
            pltpu.async_copy(hA_hbm.at[sidx[q]], rows[q], gs[q])

        @pl.when(c == 1)
        def _():
            pltpu.async_copy(hB_hbm.at[sidx[q]], rows[q], gs[q])

    def wait_gather(q):
        pltpu.make_async_copy(hA_hbm.at[sidx[q]], rows[q], gs[q]).wait()

    def issue_sidx(g, q):
        off = pl.multiple_of(base + g * K, 8)
        pltpu.async_copy(src_hbm.at[pl.ds(off, K)], sidx[q], si[q])

    def issue_didx(g, q):
        off = pl.multiple_of(base + g * K, 8)
        pltpu.async_copy(dst_hbm.at[pl.ds(off, K)], didx[q], di[q])

    def wait_sidx(q):
        pltpu.make_async_copy(src_hbm.at[pl.ds(0, K)], sidx[q], si[q]).wait()

    def wait_didx(q):
        pltpu.make_async_copy(dst_hbm.at[pl.ds(0, K)], didx[q], di[q]).wait()

    def scatter(q):
        pltpu.async_copy(rows[q], acc.at[didx[q]], sc[q], add=True)

    def wait_scatter(q):
        pltpu.make_async_copy(rows[q], acc.at[didx[q]], sc[q]).wait()

    for q in range(3):
        issue_sidx(q, q)
    issue_didx(0, 0)
    issue_didx(1, 1)
    wait_sidx(0)
    gather(0)
    wait_sidx(1)
    gather(1)

    @pl.when(c == 0)
    def _():
        pltpu.sync_copy(hA_hbm.at[pl.ds(r0, RPT)], acc.at[pl.ds(r0, RPT)])

    @pl.when(c == 1)
    def _():
        pltpu.sync_copy(hB_hbm.at[pl.ds(r0, RPT)], acc.at[pl.ds(r0, RPT)])

    plsc.subcore_barrier()

    def triple(i, carry):
        for q in range(3):
            g = 3 * i + q
            q2 = (q + 2) % 3

            @pl.when(g < CHUNKS)
            def _():
                wait_gather(q)

            @pl.when((g >= 1) & (g + 2 < CHUNKS))
            def _():
                wait_scatter(q2)

            @pl.when(g + 2 < CHUNKS)
            def _():
                issue_didx(g + 2, q2)
                wait_sidx(q2)
                gather(q2)

            @pl.when(g + 3 < CHUNKS)
            def _():
                issue_sidx(g + 3, q)

            @pl.when(g < CHUNKS)
            def _():
                wait_didx(q)
                scatter(q)

        return carry

    lax.fori_loop(0, (CHUNKS + 2) // 3, triple, 0)
    for q in range(3):
        wait_scatter(q)

    toff = base + CHUNKS * K
    pltpu.sync_copy(src_hbm.at[pl.ds(toff, TAIL)], tsidx)
    pltpu.sync_copy(dst_hbm.at[pl.ds(toff, TAIL)], tdidx)
    trows = rows0.at[pl.ds(0, TAIL)]

    @pl.when(c == 0)
    def _():
        pltpu.async_copy(hA_hbm.at[tsidx], trows, gs[0])

    @pl.when(c == 1)
    def _():
        pltpu.async_copy(hB_hbm.at[tsidx], trows, gs[0])

    pltpu.make_async_copy(hA_hbm.at[tsidx], trows, gs[0]).wait()
    pltpu.sync_copy(trows, acc.at[tdidx], add=True)
    plsc.subcore_barrier()

    @pl.when(c == 0)
    def _():
        pltpu.sync_copy(acc.at[pl.ds(r0, RPT)], outA_hbm.at[pl.ds(r0, RPT)])

    @pl.when(c == 1)
    def _():
        pltpu.sync_copy(acc.at[pl.ds(r0, RPT)], outB_hbm.at[pl.ds(r0, RPT)])


_agg_call = pl.kernel(
    _agg_body,
    out_type=[jax.ShapeDtypeStruct((NP, H), f32)] * 2,
    mesh=_mesh,
    scratch_types=(
        [pltpu.VMEM_SHARED((NP, H), f32)]
        + [pltpu.VMEM((K,), i32)] * 6
        + [pltpu.VMEM((K, H), f32)] * 3
        + [pltpu.VMEM((TAIL,), i32)] * 2
        + [pltpu.SemaphoreType.DMA] * 12
    ),
)


RB = 1000


def _dinv(degA, degB):
    return lax.rsqrt(degA[:, :1] + degB[:, :1] + 1.0)


def _mm_body(x_ref, w_ref, h_ref):
    h_ref[...] = jnp.dot(x_ref[...], w_ref[...], preferred_element_type=f32)


def _scale_body(h_ref, degA_ref, degB_ref, hA_ref, hB_ref, dinv_ref):
    dinv = _dinv(degA_ref[...], degB_ref[...])
    h = h_ref[...] * dinv
    hA_ref[...] = h[:, :H]
    hB_ref[...] = h[:, H:]
    dinv_ref[...] = dinv


def _norm(accA, accB, dinv, b, g, be):
    z = jnp.concatenate([accA, accB], axis=1) * dinv + b
    mu = jnp.mean(z, axis=-1, keepdims=True)
    var = jnp.mean((z - mu) ** 2, axis=-1, keepdims=True)
    z = (z - mu) * lax.rsqrt(var + EPS) * g + be
    return jnp.maximum(z, 0.0)


def _stage_body(accA_ref, accB_ref, dinv_ref, b_ref, g_ref, be_ref,
                w_ref, hA_ref, hB_ref):
    dinv = dinv_ref[...]
    z = _norm(accA_ref[...], accB_ref[...], dinv,
              b_ref[...], g_ref[...], be_ref[...])
    h = jnp.dot(z, w_ref[...], preferred_element_type=f32) * dinv
    hA_ref[...] = h[:, :H]
    hB_ref[...] = h[:, H:]


def _final_body(accA_ref, accB_ref, dinv_ref, b_ref, g_ref, be_ref,
                out_ref):
    out_ref[...] = _norm(accA_ref[...], accB_ref[...], dinv_ref[...],
                         b_ref[...], g_ref[...], be_ref[...])


def _rows(shape):
    return pl.BlockSpec(shape, lambda i: (i, 0))


def _whole(shape):
    return pl.BlockSpec(shape, lambda i: (0, 0))


_halves = [pl.BlockSpec((RB, H), lambda i: (i, 0))] * 2
_half_shapes = [jax.ShapeDtypeStruct((NP, H), f32)] * 2
_deg_specs = [_rows((RB, 128))] * 2
_vec_specs = [_whole((1, D))] * 3

_dinv_spec = _rows((RB, 1))

_mm_call = pl.pallas_call(
    _mm_body,
    grid=(N // RB,),
    in_specs=[_rows((RB, D)), _whole((D, D))],
    out_specs=_rows((RB, D)),
    out_shape=jax.ShapeDtypeStruct((N, D), f32),
)

_scale_call = pl.pallas_call(
    _scale_body,
    grid=(N // RB,),
    in_specs=[_rows((RB, D))] + _deg_specs,
    out_specs=_halves + [_dinv_spec],
    out_shape=_half_shapes + [jax.ShapeDtypeStruct((NP, 1), f32)],
)

_stage_call = pl.pallas_call(
    _stage_body,
    grid=(N // RB,),
    in_specs=[_rows((RB, H))] * 2 + [_dinv_spec] + _vec_specs
    + [_whole((D, D))],
    out_specs=_halves,
    out_shape=_half_shapes,
)

_final_call = pl.pallas_call(
    _final_body,
    grid=(N // RB,),
    in_specs=[_rows((RB, H))] * 2 + [_dinv_spec] + _vec_specs,
    out_specs=_rows((RB, D)),
    out_shape=jax.ShapeDtypeStruct((N, D), f32),
)


def kernel(x, edge_index, W1, b1, g1, be1, W2, b2, g2, be2, W3, b3, g3, be3):
    src = edge_index[0]
    dst = edge_index[1]
    zeros128 = jnp.zeros((NP, 128), f32)
    ones128 = jnp.ones((DK, 128), f32)

    degA, degB = _deg_call(dst, zeros128, ones128)
    h1 = _mm_call(x, W1)
    hA, hB, dinv = _scale_call(h1, degA, degB)
    accA, accB = _agg_call(hA, hB, src, dst)
    hA, hB = _stage_call(accA, accB, dinv,
                         b1.reshape(1, D), g1.reshape(1, D), be1.reshape(1, D),
                         W2)
    accA, accB = _agg_call(hA, hB, src, dst)
    hA, hB = _stage_call(accA, accB, dinv,
                         b2.reshape(1, D), g2.reshape(1, D), be2.reshape(1, D),
                         W3)
    accA, accB = _agg_call(hA, hB, src, dst)
    return _final_call(accA, accB, dinv,
                       b3.reshape(1, D), g3.reshape(1, D), be3.reshape(1, D))


# speedup vs baseline: 1.1030x; 1.1030x over previous
import jax
import jax.numpy as jnp
from jax import lax
from jax.experimental import pallas as pl
from jax.experimental.pallas import tpu as pltpu
from jax.experimental.pallas import tpu_sc as plsc

N = 10000
E = 160000
D = 256
H = D // 2
EPS = 1e-5

NC = 2
NS = 16
NP = 10240
RPT = NP // NS
EPT = E // NS
K = 120
CHUNKS = EPT // K
TAIL = EPT - CHUNKS * K

DEG_E_SC = E // NC
DEG_E_T = DEG_E_SC // NS
DK = 128
DEG_CHUNKS = DEG_E_T // DK
DEG_TAIL = DEG_E_T - DEG_CHUNKS * DK

_mesh = plsc.VectorSubcoreMesh(core_axis_name="c", subcore_axis_name="s")
f32 = jnp.float32
i32 = jnp.int32


def _deg_body(dst_hbm, zeros_hbm, ones_hbm, degA_hbm, degB_hbm,
              acc, ones_v,
              didx0, didx1, didx2, didx3, didx4, didx5, didx6, didx7, tdidx,
              di0, di1, di2, di3, di4, di5, di6, di7,
              sc0, sc1, sc2, sc3):
    c = lax.axis_index("c")
    s = lax.axis_index("s")
    r0 = s * RPT
    base = c * DEG_E_SC + s * DEG_E_T
    didx = (didx0, didx1, didx2, didx3, didx4, didx5, didx6, didx7)
    di = (di0, di1, di2, di3, di4, di5, di6, di7)
    sc = (sc0, sc1, sc2, sc3)

    def issue_idx(g, q):
        off = pl.multiple_of(base + g * DK, 8)
        pltpu.async_copy(dst_hbm.at[pl.ds(off, DK)], didx[q], di[q])

    def wait_idx(q):
        pltpu.make_async_copy(dst_hbm.at[pl.ds(0, DK)], didx[q], di[q]).wait()

    def scatter(q8, q4):
        pltpu.async_copy(ones_v, acc.at[didx[q8]], sc[q4], add=True)

    def wait_scatter(q4):
        pltpu.make_async_copy(ones_v, acc.at[didx[0]], sc[q4]).wait()

    for q in range(4):
        issue_idx(q, q)
    pltpu.sync_copy(ones_hbm, ones_v)
    pltpu.sync_copy(zeros_hbm.at[pl.ds(r0, RPT)], acc.at[pl.ds(r0, RPT)])
    plsc.subcore_barrier()

    def octet(i, carry):
        for j in range(8):
            g = 8 * i + j
            q4 = j % 4

            @pl.when(g < DEG_CHUNKS)
            def _():
                wait_idx(j)

            @pl.when((g >= 4) & (g < DEG_CHUNKS))
            def _():
                wait_scatter(q4)

            @pl.when(g < DEG_CHUNKS)
            def _():
                scatter(j, q4)

            @pl.when(g + 4 < DEG_CHUNKS)
            def _():
                issue_idx(g + 4, (j + 4) % 8)

        return carry

    lax.fori_loop(0, (DEG_CHUNKS + 7) // 8, octet, 0)
    for q4 in range(4):
        wait_scatter(q4)
    toff = base + DEG_CHUNKS * DK
    pltpu.sync_copy(dst_hbm.at[pl.ds(toff, DEG_TAIL)], tdidx)
    pltpu.sync_copy(ones_v.at[pl.ds(0, DEG_TAIL)], acc.at[tdidx], add=True)
    plsc.subcore_barrier()

    @pl.when(c == 0)
    def _():
        pltpu.sync_copy(acc.at[pl.ds(r0, RPT)], degA_hbm.at[pl.ds(r0, RPT)])

    @pl.when(c == 1)
    def _():
        pltpu.sync_copy(acc.at[pl.ds(r0, RPT)], degB_hbm.at[pl.ds(r0, RPT)])


_deg_call = pl.kernel(
    _deg_body,
    out_type=[jax.ShapeDtypeStruct((NP,), f32)] * 2,
    mesh=_mesh,
    scratch_types=(
        [pltpu.VMEM_SHARED((NP,), f32)]
        + [pltpu.VMEM((DK,), f32)]
        + [pltpu.VMEM((DK,), i32)] * 8
        + [pltpu.VMEM((DEG_TAIL,), i32)]
        + [pltpu.SemaphoreType.DMA] * 12
    ),
)


def _agg_body(hA_hbm, hB_hbm, src_hbm, dst_hbm, outA_hbm, outB_hbm,
              acc, sidx0, sidx1, sidx2, didx0, didx1, didx2,
              rows0, rows1, rows2, tsidx, tdidx,
              si0, si1, si2, di0, di1, di2,
              g0, g1, g2, sc0, sc1, sc2):
    c = lax.axis_index("c")
    s = lax.axis_index("s")
    r0 = s * RPT
    base = s * EPT
    sidx = (sidx0, sidx1, sidx2)
    didx = (didx0, didx1, didx2)
    rows = (rows0, rows1, rows2)
    si = (si0, si1, si2)
    di = (di0, di1, di2)
    gs = (g0, g1, g2)
    sc = (sc0, sc1, sc2)

    def gather(q):
        @pl.when(c == 0)
        def _():
            pltpu.async_copy(hA_hbm.at[sidx[q]], rows[q], gs[q])

        @pl.when(c == 1)
        def _():
            pltpu.async_copy(hB_hbm.at[sidx[q]], rows[q], gs[q])

    def wait_gather(q):
        pltpu.make_async_copy(hA_hbm.at[sidx[q]], rows[q], gs[q]).wait()

    def issue_sidx(g, q):
        off = pl.multiple_of(base + g * K, 8)
        pltpu.async_copy(src_hbm.at[pl.ds(off, K)], sidx[q], si[q])

    def issue_didx(g, q):
        off = pl.multiple_of(base + g * K, 8)
        pltpu.async_copy(dst_hbm.at[pl.ds(off, K)], didx[q], di[q])

    def wait_sidx(q):
        pltpu.make_async_copy(src_hbm.at[pl.ds(0, K)], sidx[q], si[q]).wait()

    def wait_didx(q):
        pltpu.make_async_copy(dst_hbm.at[pl.ds(0, K)], didx[q], di[q]).wait()

    def scatter(q):
        pltpu.async_copy(rows[q], acc.at[didx[q]], sc[q], add=True)

    def wait_scatter(q):
        pltpu.make_async_copy(rows[q], acc.at[didx[q]], sc[q]).wait()

    for q in range(3):
        issue_sidx(q, q)
    issue_didx(0, 0)
    issue_didx(1, 1)
    wait_sidx(0)
    gather(0)
    wait_sidx(1)
    gather(1)

    @pl.when(c == 0)
    def _():
        pltpu.sync_copy(hA_hbm.at[pl.ds(r0, RPT)], acc.at[pl.ds(r0, RPT)])

    @pl.when(c == 1)
    def _():
        pltpu.sync_copy(hB_hbm.at[pl.ds(r0, RPT)], acc.at[pl.ds(r0, RPT)])

    plsc.subcore_barrier()

    def triple(i, carry):
        for q in range(3):
            g = 3 * i + q
            q2 = (q + 2) % 3

            @pl.when(g < CHUNKS)
            def _():
                wait_gather(q)

            @pl.when((g >= 1) & (g + 2 < CHUNKS))
            def _():
                wait_scatter(q2)

            @pl.when(g + 2 < CHUNKS)
            def _():
                issue_didx(g + 2, q2)
                wait_sidx(q2)
                gather(q2)

            @pl.when(g + 3 < CHUNKS)
            def _():
                issue_sidx(g + 3, q)

            @pl.when(g < CHUNKS)
            def _():
                wait_didx(q)
                scatter(q)

        return carry

    lax.fori_loop(0, (CHUNKS + 2) // 3, triple, 0)
    for q in range(3):
        wait_scatter(q)

    toff = base + CHUNKS * K
    pltpu.sync_copy(src_hbm.at[pl.ds(toff, TAIL)], tsidx)
    pltpu.sync_copy(dst_hbm.at[pl.ds(toff, TAIL)], tdidx)
    trows = rows0.at[pl.ds(0, TAIL)]

    @pl.when(c == 0)
    def _():
        pltpu.async_copy(hA_hbm.at[tsidx], trows, gs[0])

    @pl.when(c == 1)
    def _():
        pltpu.async_copy(hB_hbm.at[tsidx], trows, gs[0])

    pltpu.make_async_copy(hA_hbm.at[tsidx], trows, gs[0]).wait()
    pltpu.sync_copy(trows, acc.at[tdidx], add=True)
    plsc.subcore_barrier()

    @pl.when(c == 0)
    def _():
        pltpu.sync_copy(acc.at[pl.ds(r0, RPT)], outA_hbm.at[pl.ds(r0, RPT)])

    @pl.when(c == 1)
    def _():
        pltpu.sync_copy(acc.at[pl.ds(r0, RPT)], outB_hbm.at[pl.ds(r0, RPT)])


_agg_call = pl.kernel(
    _agg_body,
    out_type=[jax.ShapeDtypeStruct((NP, H), f32)] * 2,
    mesh=_mesh,
    scratch_types=(
        [pltpu.VMEM_SHARED((NP, H), f32)]
        + [pltpu.VMEM((K,), i32)] * 6
        + [pltpu.VMEM((K, H), f32)] * 3
        + [pltpu.VMEM((TAIL,), i32)] * 2
        + [pltpu.SemaphoreType.DMA] * 12
    ),
)


RB = 1000
PRB = 1024


def _pre_body(x_ref, w_ref, degA_ref, degB_ref, hA_ref, hB_ref, dinv_ref):
    i = pl.program_id(0)
    dA = degA_ref[pl.ds(i * PRB, PRB)]
    dB = degB_ref[pl.ds(i * PRB, PRB)]
    dinv = jnp.reshape(lax.rsqrt(dA + dB + 1.0), (PRB, 1))
    h = jnp.dot(x_ref[...], w_ref[...], preferred_element_type=f32) * dinv
    hA_ref[...] = h[:, :H]
    hB_ref[...] = h[:, H:]
    dinv_ref[...] = dinv


def _norm(accA, accB, dinv, b, g, be):
    z = jnp.concatenate([accA, accB], axis=1) * dinv + b
    mu = jnp.mean(z, axis=-1, keepdims=True)
    var = jnp.mean((z - mu) ** 2, axis=-1, keepdims=True)
    z = (z - mu) * lax.rsqrt(var + EPS) * g + be
    return jnp.maximum(z, 0.0)


def _stage_body(accA_ref, accB_ref, dinv_ref, b_ref, g_ref, be_ref,
                w_ref, hA_ref, hB_ref):
    dinv = dinv_ref[...]
    z = _norm(accA_ref[...], accB_ref[...], dinv,
              b_ref[...], g_ref[...], be_ref[...])
    h = jnp.dot(z, w_ref[...], preferred_element_type=f32) * dinv
    hA_ref[...] = h[:, :H]
    hB_ref[...] = h[:, H:]


def _final_body(accA_ref, accB_ref, dinv_ref, b_ref, g_ref, be_ref,
                out_ref):
    out_ref[...] = _norm(accA_ref[...], accB_ref[...], dinv_ref[...],
                         b_ref[...], g_ref[...], be_ref[...])


def _rows(shape):
    return pl.BlockSpec(shape, lambda i: (i, 0))


def _whole(shape):
    return pl.BlockSpec(shape, lambda i: (0, 0))


_halves = [pl.BlockSpec((RB, H), lambda i: (i, 0))] * 2
_half_shapes = [jax.ShapeDtypeStruct((NP, H), f32)] * 2
_vec_specs = [_whole((1, D))] * 3

_dinv_spec = _rows((RB, 1))
_deg1d_specs = [pl.BlockSpec((NP,), lambda i: (0,))] * 2

_pre_call = pl.pallas_call(
    _pre_body,
    grid=(NP // PRB,),
    in_specs=[_rows((PRB, D)), _whole((D, D))] + _deg1d_specs,
    out_specs=[pl.BlockSpec((PRB, H), lambda i: (i, 0))] * 2
    + [pl.BlockSpec((PRB, 1), lambda i: (i, 0))],
    out_shape=_half_shapes + [jax.ShapeDtypeStruct((NP, 1), f32)],
)

_stage_call = pl.pallas_call(
    _stage_body,
    grid=(N // RB,),
    in_specs=[_rows((RB, H))] * 2 + [_dinv_spec] + _vec_specs
    + [_whole((D, D))],
    out_specs=_halves,
    out_shape=_half_shapes,
)

_final_call = pl.pallas_call(
    _final_body,
    grid=(N // RB,),
    in_specs=[_rows((RB, H))] * 2 + [_dinv_spec] + _vec_specs,
    out_specs=_rows((RB, D)),
    out_shape=jax.ShapeDtypeStruct((N, D), f32),
)


def kernel(x, edge_index, W1, b1, g1, be1, W2, b2, g2, be2, W3, b3, g3, be3):
    src = edge_index[0]
    dst = edge_index[1]
    zeros1d = jnp.zeros((NP,), f32)
    ones1d = jnp.ones((DK,), f32)

    degA, degB = _deg_call(dst, zeros1d, ones1d)
    xp = jnp.pad(x, ((0, NP - N), (0, 0)))
    hA, hB, dinv = _pre_call(xp, W1, degA, degB)
    accA, accB = _agg_call(hA, hB, src, dst)
    hA, hB = _stage_call(accA, accB, dinv,
                         b1.reshape(1, D), g1.reshape(1, D), be1.reshape(1, D),
                         W2)
    accA, accB = _agg_call(hA, hB, src, dst)
    hA, hB = _stage_call(accA, accB, dinv,
                         b2.reshape(1, D), g2.reshape(1, D), be2.reshape(1, D),
                         W3)
    accA, accB = _agg_call(hA, hB, src, dst)
    return _final_call(accA, accB, dinv,
                       b3.reshape(1, D), g3.reshape(1, D), be3.reshape(1, D))


# speedup vs baseline: 1.1034x; 1.0004x over previous
import jax
import jax.numpy as jnp
from jax import lax
from jax.experimental import pallas as pl
from jax.experimental.pallas import tpu as pltpu
from jax.experimental.pallas import tpu_sc as plsc

N = 10000
E = 160000
D = 256
H = D // 2
EPS = 1e-5

NC = 2
NS = 16
NP = 10240
RPT = NP // NS
EPT = E // NS
K = 120
CHUNKS = EPT // K
TAIL = EPT - CHUNKS * K

DEG_E_SC = E // NC
DEG_E_T = DEG_E_SC // NS
DK = 128
DEG_CHUNKS = DEG_E_T // DK
DEG_TAIL = DEG_E_T - DEG_CHUNKS * DK

_mesh = plsc.VectorSubcoreMesh(core_axis_name="c", subcore_axis_name="s",
                               num_cores=NC, num_subcores=NS)
f32 = jnp.float32
i32 = jnp.int32


def _deg_body(dst_hbm, zeros_hbm, ones_hbm, degA_hbm, degB_hbm,
              acc, ones_v,
              didx0, didx1, didx2, didx3, didx4, didx5, didx6, didx7, tdidx,
              di0, di1, di2, di3, di4, di5, di6, di7,
              sc0, sc1, sc2, sc3):
    c = lax.axis_index("c")
    s = lax.axis_index("s")
    r0 = s * RPT
    base = c * DEG_E_SC + s * DEG_E_T
    didx = (didx0, didx1, didx2, didx3, didx4, didx5, didx6, didx7)
    di = (di0, di1, di2, di3, di4, di5, di6, di7)
    sc = (sc0, sc1, sc2, sc3)

    def issue_idx(g, q):
        off = pl.multiple_of(base + g * DK, 8)
        pltpu.async_copy(dst_hbm.at[pl.ds(off, DK)], didx[q], di[q])

    def wait_idx(q):
        pltpu.make_async_copy(dst_hbm.at[pl.ds(0, DK)], didx[q], di[q]).wait()

    def scatter(q8, q4):
        pltpu.async_copy(ones_v, acc.at[didx[q8]], sc[q4], add=True)

    def wait_scatter(q4):
        pltpu.make_async_copy(ones_v, acc.at[didx[0]], sc[q4]).wait()

    for q in range(4):
        issue_idx(q, q)
    pltpu.sync_copy(ones_hbm, ones_v)
    pltpu.sync_copy(zeros_hbm.at[pl.ds(r0, RPT)], acc.at[pl.ds(r0, RPT)])
    plsc.subcore_barrier()

    def octet(i, carry):
        for j in range(8):
            g = 8 * i + j
            q4 = j % 4

            @pl.when(g < DEG_CHUNKS)
            def _():
                wait_idx(j)

            @pl.when((g >= 4) & (g < DEG_CHUNKS))
            def _():
                wait_scatter(q4)

            @pl.when(g < DEG_CHUNKS)
            def _():
                scatter(j, q4)

            @pl.when(g + 4 < DEG_CHUNKS)
            def _():
                issue_idx(g + 4, (j + 4) % 8)

        return carry

    lax.fori_loop(0, (DEG_CHUNKS + 7) // 8, octet, 0)
    for q4 in range(4):
        wait_scatter(q4)
    toff = base + DEG_CHUNKS * DK
    pltpu.sync_copy(dst_hbm.at[pl.ds(toff, DEG_TAIL)], tdidx)
    pltpu.sync_copy(ones_v.at[pl.ds(0, DEG_TAIL)], acc.at[tdidx], add=True)
    plsc.subcore_barrier()

    @pl.when(c == 0)
    def _():
        pltpu.sync_copy(acc.at[pl.ds(r0, RPT)], degA_hbm.at[pl.ds(r0, RPT)])

    @pl.when(c == 1)
    def _():
        pltpu.sync_copy(acc.at[pl.ds(r0, RPT)], degB_hbm.at[pl.ds(r0, RPT)])


_deg_call = pl.kernel(
    _deg_body,
    out_type=[jax.ShapeDtypeStruct((NP,), f32)] * 2,
    mesh=_mesh,
    scratch_types=(
        [pltpu.VMEM_SHARED((NP,), f32)]
        + [pltpu.VMEM((DK,), f32)]
        + [pltpu.VMEM((DK,), i32)] * 8
        + [pltpu.VMEM((DEG_TAIL,), i32)]
        + [pltpu.SemaphoreType.DMA] * 12
    ),
)


def _agg_body(hA_hbm, hB_hbm, src_hbm, dst_hbm, outA_hbm, outB_hbm,
              acc, sidx0, sidx1, sidx2, didx0, didx1, didx2,
              rows0, rows1, rows2, tsidx, tdidx,
              si0, si1, si2, di0, di1, di2,
              g0, g1, g2, sc0, sc1, sc2):
    c = lax.axis_index("c")
    s = lax.axis_index("s")
    r0 = s * RPT
    base = s * EPT
    sidx = (sidx0, sidx1, sidx2)
    didx = (didx0, didx1, didx2)
    rows = (rows0, rows1, rows2)
    si = (si0, si1, si2)
    di = (di0, di1, di2)
    gs = (g0, g1, g2)
    sc = (sc0, sc1, sc2)

    def gather(q):
        @pl.when(c == 0)
        def _():
            pltpu.async_copy(hA_hbm.at[sidx[q]], rows[q], gs[q])

        @pl.when(c == 1)
        def _():
            pltpu.async_copy(hB_hbm.at[sidx[q]], rows[q], gs[q])

    def wait_gather(q):
        pltpu.make_async_copy(hA_hbm.at[sidx[q]], rows[q], gs[q]).wait()

    def issue_sidx(g, q):
        off = pl.multiple_of(base + g * K, 8)
        pltpu.async_copy(src_hbm.at[pl.ds(off, K)], sidx[q], si[q])

    def issue_didx(g, q):
        off = pl.multiple_of(base + g * K, 8)
        pltpu.async_copy(dst_hbm.at[pl.ds(off, K)], didx[q], di[q])

    def wait_sidx(q):
        pltpu.make_async_copy(src_hbm.at[pl.ds(0, K)], sidx[q], si[q]).wait()

    def wait_didx(q):
        pltpu.make_async_copy(dst_hbm.at[pl.ds(0, K)], didx[q], di[q]).wait()

    def scatter(q):
        pltpu.async_copy(rows[q], acc.at[didx[q]], sc[q], add=True)

    def wait_scatter(q):
        pltpu.make_async_copy(rows[q], acc.at[didx[q]], sc[q]).wait()

    for q in range(3):
        issue_sidx(q, q)
    issue_didx(0, 0)
    issue_didx(1, 1)
    wait_sidx(0)
    gather(0)
    wait_sidx(1)
    gather(1)

    @pl.when(c == 0)
    def _():
        pltpu.sync_copy(hA_hbm.at[pl.ds(r0, RPT)], acc.at[pl.ds(r0, RPT)])

    @pl.when(c == 1)
    def _():
        pltpu.sync_copy(hB_hbm.at[pl.ds(r0, RPT)], acc.at[pl.ds(r0, RPT)])

    plsc.subcore_barrier()

    def triple(i, carry):
        for q in range(3):
            g = 3 * i + q
            q2 = (q + 2) % 3

            @pl.when(g < CHUNKS)
            def _():
                wait_gather(q)

            @pl.when((g >= 1) & (g + 2 < CHUNKS))
            def _():
                wait_scatter(q2)

            @pl.when(g + 2 < CHUNKS)
            def _():
                issue_didx(g + 2, q2)
                wait_sidx(q2)
                gather(q2)

            @pl.when(g + 3 < CHUNKS)
            def _():
                issue_sidx(g + 3, q)

            @pl.when(g < CHUNKS)
            def _():
                wait_didx(q)
                scatter(q)

        return carry

    lax.fori_loop(0, (CHUNKS + 2) // 3, triple, 0)
    for q in range(3):
        wait_scatter(q)

    toff = base + CHUNKS * K
    pltpu.sync_copy(src_hbm.at[pl.ds(toff, TAIL)], tsidx)
    pltpu.sync_copy(dst_hbm.at[pl.ds(toff, TAIL)], tdidx)
    trows = rows0.at[pl.ds(0, TAIL)]

    @pl.when(c == 0)
    def _():
        pltpu.async_copy(hA_hbm.at[tsidx], trows, gs[0])

    @pl.when(c == 1)
    def _():
        pltpu.async_copy(hB_hbm.at[tsidx], trows, gs[0])

    pltpu.make_async_copy(hA_hbm.at[tsidx], trows, gs[0]).wait()
    pltpu.sync_copy(trows, acc.at[tdidx], add=True)
    plsc.subcore_barrier()

    @pl.when(c == 0)
    def _():
        pltpu.sync_copy(acc.at[pl.ds(r0, RPT)], outA_hbm.at[pl.ds(r0, RPT)])

    @pl.when(c == 1)
    def _():
        pltpu.sync_copy(acc.at[pl.ds(r0, RPT)], outB_hbm.at[pl.ds(r0, RPT)])


_agg_call = pl.kernel(
    _agg_body,
    out_type=[jax.ShapeDtypeStruct((NP, H), f32)] * 2,
    mesh=_mesh,
    scratch_types=(
        [pltpu.VMEM_SHARED((NP, H), f32)]
        + [pltpu.VMEM((K,), i32)] * 6
        + [pltpu.VMEM((K, H), f32)] * 3
        + [pltpu.VMEM((TAIL,), i32)] * 2
        + [pltpu.SemaphoreType.DMA] * 12
    ),
)


RB = 1000
PRB = 1024


def _pre_body(x_ref, w_ref, degA_ref, degB_ref, hA_ref, hB_ref, dinv_ref):
    i = pl.program_id(0)
    dA = degA_ref[pl.ds(i * PRB, PRB)]
    dB = degB_ref[pl.ds(i * PRB, PRB)]
    dinv = jnp.reshape(lax.rsqrt(dA + dB + 1.0), (PRB, 1))
    h = jnp.dot(x_ref[...], w_ref[...], preferred_element_type=f32) * dinv
    hA_ref[...] = h[:, :H]
    hB_ref[...] = h[:, H:]
    dinv_ref[...] = dinv


def _norm(accA, accB, dinv, b, g, be):
    z = jnp.concatenate([accA, accB], axis=1) * dinv + b
    mu = jnp.mean(z, axis=-1, keepdims=True)
    var = jnp.mean((z - mu) ** 2, axis=-1, keepdims=True)
    z = (z - mu) * lax.rsqrt(var + EPS) * g + be
    return jnp.maximum(z, 0.0)


def _stage_body(accA_ref, accB_ref, dinv_ref, b_ref, g_ref, be_ref,
                w_ref, hA_ref, hB_ref):
    dinv = dinv_ref[...]
    z = _norm(accA_ref[...], accB_ref[...], dinv,
              b_ref[...], g_ref[...], be_ref[...])
    h = jnp.dot(z, w_ref[...], preferred_element_type=f32) * dinv
    hA_ref[...] = h[:, :H]
    hB_ref[...] = h[:, H:]


def _final_body(accA_ref, accB_ref, dinv_ref, b_ref, g_ref, be_ref,
                out_ref):
    out_ref[...] = _norm(accA_ref[...], accB_ref[...], dinv_ref[...],
                         b_ref[...], g_ref[...], be_ref[...])


def _rows(shape):
    return pl.BlockSpec(shape, lambda i: (i, 0))


def _whole(shape):
    return pl.BlockSpec(shape, lambda i: (0, 0))


_halves = [pl.BlockSpec((RB, H), lambda i: (i, 0))] * 2
_half_shapes = [jax.ShapeDtypeStruct((NP, H), f32)] * 2
_vec_specs = [_whole((1, D))] * 3

_dinv_spec = _rows((RB, 1))
_deg1d_specs = [pl.BlockSpec((NP,), lambda i: (0,))] * 2

_pre_call = pl.pallas_call(
    _pre_body,
    grid=(NP // PRB,),
    in_specs=[_rows((PRB, D)), _whole((D, D))] + _deg1d_specs,
    out_specs=[pl.BlockSpec((PRB, H), lambda i: (i, 0))] * 2
    + [pl.BlockSpec((PRB, 1), lambda i: (i, 0))],
    out_shape=_half_shapes + [jax.ShapeDtypeStruct((NP, 1), f32)],
)

_stage_call = pl.pallas_call(
    _stage_body,
    grid=(N // RB,),
    in_specs=[_rows((RB, H))] * 2 + [_dinv_spec] + _vec_specs
    + [_whole((D, D))],
    out_specs=_halves,
    out_shape=_half_shapes,
)

_final_call = pl.pallas_call(
    _final_body,
    grid=(N // RB,),
    in_specs=[_rows((RB, H))] * 2 + [_dinv_spec] + _vec_specs,
    out_specs=_rows((RB, D)),
    out_shape=jax.ShapeDtypeStruct((N, D), f32),
)


def kernel(x, edge_index, W1, b1, g1, be1, W2, b2, g2, be2, W3, b3, g3, be3):
    src = edge_index[0]
    dst = edge_index[1]
    zeros1d = jnp.zeros((NP,), f32)
    ones1d = jnp.ones((DK,), f32)

    degA, degB = _deg_call(dst, zeros1d, ones1d)
    xp = jnp.pad(x, ((0, NP - N), (0, 0)))
    hA, hB, dinv = _pre_call(xp, W1, degA, degB)
    accA, accB = _agg_call(hA, hB, src, dst)
    hA, hB = _stage_call(accA, accB, dinv,
                         b1.reshape(1, D), g1.reshape(1, D), be1.reshape(1, D),
                         W2)
    accA, accB = _agg_call(hA, hB, src, dst)
    hA, hB = _stage_call(accA, accB, dinv,
                         b2.reshape(1, D), g2.reshape(1, D), be2.reshape(1, D),
                         W3)
    accA, accB = _agg_call(hA, hB, src, dst)
    return _final_call(accA, accB, dinv,
                       b3.reshape(1, D), g3.reshape(1, D), be3.reshape(1, D))
